# baseline (speedup 1.0000x reference)
import functools

import jax
import jax.numpy as jnp
from jax import lax
from jax.experimental import pallas as pl
from jax.experimental.pallas import tpu as pltpu
from jax.experimental.pallas import tpu_sc as plsc

F32 = jnp.float32
NN = 2048
NE = 65536
HID = 128
EMB = 32
NLAYERS = 4
NC, NS = 2, 16
NW = NC * NS
EPT = NE // NW
CH = 128

_mesh = functools.partial(
    plsc.VectorSubcoreMesh, core_axis_name="c", subcore_axis_name="s")


def _sc_gather_add(tab_a, tab_b, row, col):
  ne = row.shape[0]
  ept = ne // NW
  nch = ept // CH

  @functools.partial(
      pl.kernel,
      out_type=jax.ShapeDtypeStruct((ne, HID), F32),
      mesh=_mesh(),
      scratch_types=[
          pltpu.VMEM((nch, CH), jnp.int32), pltpu.VMEM((nch, CH), jnp.int32),
          pltpu.VMEM((CH, HID), F32), pltpu.VMEM((CH, HID), F32),
          pltpu.VMEM((CH, HID), F32), pltpu.VMEM((CH, HID), F32),
          pltpu.VMEM((CH, HID), F32), pltpu.VMEM((CH, HID), F32),
          pltpu.SemaphoreType.DMA, pltpu.SemaphoreType.DMA,
          pltpu.SemaphoreType.DMA, pltpu.SemaphoreType.DMA,
          pltpu.SemaphoreType.DMA, pltpu.SemaphoreType.DMA,
      ],
  )
  def k(a_hbm, b_hbm, row_hbm, col_hbm, s_hbm,
        idxr, idxc, bufa0, bufa1, bufb0, bufb1,
        ubuf0, ubuf1, ga0, ga1, gb0, gb1, w0, w1):
    bufa = [bufa0, bufa1]
    bufb = [bufb0, bufb1]
    ubuf = [ubuf0, ubuf1]
    ga = [ga0, ga1]
    gb = [gb0, gb1]
    w = [w0, w1]
    wid = lax.axis_index("s") * NC + lax.axis_index("c")
    base = wid * ept

    pltpu.sync_copy(row_hbm.at[pl.ds(wid * nch, nch)], idxr)
    pltpu.sync_copy(col_hbm.at[pl.ds(wid * nch, nch)], idxc)

    def issue(i, b):
      pltpu.async_copy(a_hbm.at[idxr.at[i]], bufa[b], ga[b])
      pltpu.async_copy(b_hbm.at[idxc.at[i]], bufb[b], gb[b])

    issue(0, 0)

    def outer(j, carry):
      for b in (0, 1):
        i = 2 * j + b
        nb = 1 - b

        @pl.when(i + 1 < nch)
        def _():
          issue(i + 1, nb)

        pltpu.make_async_copy(a_hbm.at[idxr.at[0]], bufa[b], ga[b]).wait()
        pltpu.make_async_copy(b_hbm.at[idxc.at[0]], bufb[b], gb[b]).wait()

        @pl.when(i >= 2)
        def _():
          pltpu.make_async_copy(
              ubuf[b], s_hbm.at[pl.ds(base, CH)], w[b]).wait()

        def crow(r, c1):
          for c8 in range(HID // 16):
            sl = pl.ds(c8 * 16, 16)
            ubuf[b][r, sl] = bufa[b][r, sl] + bufb[b][r, sl]
          return c1

        lax.fori_loop(0, CH, crow, 0)
        pltpu.async_copy(ubuf[b], s_hbm.at[pl.ds(base + i * CH, CH)], w[b])
      return carry

    lax.fori_loop(0, nch // 2, outer, 0)
    pltpu.make_async_copy(ubuf0, s_hbm.at[pl.ds(base, CH)], w0).wait()
    pltpu.make_async_copy(ubuf1, s_hbm.at[pl.ds(base, CH)], w1).wait()

  return k(tab_a, tab_b, row.reshape(-1, CH), col.reshape(-1, CH))


def _sc_segment_sum(m, row):
  rpt = NN // NS
  ne = row.shape[0]
  ept = ne // NW

  @functools.partial(
      pl.kernel,
      out_type=jax.ShapeDtypeStruct((NC * NN, HID), F32),
      mesh=_mesh(),
      scratch_types=[
          pltpu.VMEM((ept // CH, CH), jnp.int32),
          pltpu.VMEM((CH, HID), F32), pltpu.VMEM((CH, HID), F32),
          pltpu.VMEM((rpt, HID), F32),
          pltpu.VMEM_SHARED((NN, HID), F32),
          pltpu.SemaphoreType.DMA, pltpu.SemaphoreType.DMA,
      ],
  )
  def k(m_hbm, row_hbm, out_hbm, idx, mbuf0, mbuf1, tbuf, acc_sh,
        ml0, ml1):
    mbuf = [mbuf0, mbuf1]
    ml = [ml0, ml1]
    cid = lax.axis_index("c")
    sid = lax.axis_index("s")
    wid = sid * NC + cid
    nch = ept // CH

    def zrow(r, carry):
      def zcol(c8, c2):
        tbuf[r, pl.ds(c8 * 16, 16)] = jnp.zeros((16,), F32)
        return c2
      return lax.fori_loop(0, HID // 16, zcol, carry)

    lax.fori_loop(0, rpt, zrow, 0)
    pltpu.sync_copy(tbuf, acc_sh.at[pl.ds(sid * rpt, rpt)])
    plsc.subcore_barrier()

    base = wid * ept
    pltpu.sync_copy(row_hbm.at[pl.ds(wid * nch, nch)], idx)

    def issue(i, b):
      pltpu.async_copy(m_hbm.at[pl.ds(base + i * CH, CH)], mbuf[b], ml[b])

    issue(0, 0)

    def outer(j, carry):
      for b in (0, 1):
        i = 2 * j + b
        nb = 1 - b

        @pl.when(i + 1 < nch)
        def _():
          issue(i + 1, nb)

        pltpu.make_async_copy(
            m_hbm.at[pl.ds(base, CH)], mbuf[b], ml[b]).wait()
        pltpu.sync_copy(mbuf[b], acc_sh.at[idx.at[i]], add=True)
      return carry

    lax.fori_loop(0, nch // 2, outer, 0)
    plsc.subcore_barrier()

    pltpu.sync_copy(acc_sh.at[pl.ds(sid * rpt, rpt)], tbuf)
    pltpu.sync_copy(tbuf, out_hbm.at[pl.ds(cid * NN + sid * rpt, rpt)])

  return k(m, row.reshape(-1, CH))


def _sc_gather_scatter(tab_a, tab_b, rowg, colg, m, rows):
  ne = rowg.shape[0]
  ept = ne // NW
  nch = ept // CH
  rpt = NN // NS

  @functools.partial(
      pl.kernel,
      out_type=(jax.ShapeDtypeStruct((ne, HID), F32),
                jax.ShapeDtypeStruct((NC * NN, HID), F32)),
      mesh=_mesh(),
      scratch_types=[
          pltpu.VMEM((nch, CH), jnp.int32), pltpu.VMEM((nch, CH), jnp.int32),
          pltpu.VMEM((nch, CH), jnp.int32),
          pltpu.VMEM((CH, HID), F32), pltpu.VMEM((CH, HID), F32),
          pltpu.VMEM((CH, HID), F32), pltpu.VMEM((CH, HID), F32),
          pltpu.VMEM((CH, HID), F32), pltpu.VMEM((CH, HID), F32),
          pltpu.VMEM_SHARED((NN, HID), F32),
          pltpu.SemaphoreType.DMA, pltpu.SemaphoreType.DMA,
          pltpu.SemaphoreType.DMA, pltpu.SemaphoreType.DMA,
          pltpu.SemaphoreType.DMA, pltpu.SemaphoreType.DMA,
          pltpu.SemaphoreType.DMA, pltpu.SemaphoreType.DMA,
      ],
  )
  def k(a_hbm, b_hbm, rowg_hbm, colg_hbm, m_hbm, rows_hbm, s_hbm, out_hbm,
        idxr, idxc, idxs, bufa0, bufa1, bufb0, bufb1, mbuf0, mbuf1,
        acc_sh, ga0, ga1, gb0, gb1, ml0, ml1, w0, w1):
    tbuf = bufb0
    bufa = [bufa0, bufa1]
    bufb = [bufb0, bufb1]
    mbuf = [mbuf0, mbuf1]
    ga = [ga0, ga1]
    gb = [gb0, gb1]
    ml = [ml0, ml1]
    w = [w0, w1]
    cid = lax.axis_index("c")
    sid = lax.axis_index("s")
    wid = sid * NC + cid
    base = wid * ept

    pltpu.sync_copy(rowg_hbm.at[pl.ds(wid * nch, nch)], idxr)
    pltpu.sync_copy(colg_hbm.at[pl.ds(wid * nch, nch)], idxc)
    pltpu.sync_copy(rows_hbm.at[pl.ds(wid * nch, nch)], idxs)

    def zrow(r, carry):
      def zcol(c8, c2):
        tbuf[r, pl.ds(c8 * 16, 16)] = jnp.zeros((16,), F32)
        return c2
      return lax.fori_loop(0, HID // 16, zcol, carry)

    lax.fori_loop(0, rpt, zrow, 0)
    pltpu.sync_copy(tbuf, acc_sh.at[pl.ds(sid * rpt, rpt)])
    plsc.subcore_barrier()

    def issue(i, b):
      pltpu.async_copy(a_hbm.at[idxr.at[i]], bufa[b], ga[b])
      pltpu.async_copy(b_hbm.at[idxc.at[i]], bufb[b], gb[b])
      pltpu.async_copy(m_hbm.at[pl.ds(base + i * CH, CH)], mbuf[b], ml[b])

    issue(0, 0)

    def outer(j, carry):
      for b in (0, 1):
        i = 2 * j + b
        nb = 1 - b

        @pl.when(i >= 1)
        def _():
          pltpu.make_async_copy(
              bufa[nb], s_hbm.at[pl.ds(base, CH)], w[nb]).wait()

        @pl.when(i + 1 < nch)
        def _():
          issue(i + 1, nb)

        pltpu.make_async_copy(a_hbm.at[idxr.at[0]], bufa[b], ga[b]).wait()
        pltpu.make_async_copy(b_hbm.at[idxc.at[0]], bufb[b], gb[b]).wait()

        def crow(r, c1):
          for c8 in range(HID // 16):
            sl = pl.ds(c8 * 16, 16)
            bufa[b][r, sl] = bufa[b][r, sl] + bufb[b][r, sl]
          return c1

        lax.fori_loop(0, CH, crow, 0)
        pltpu.async_copy(bufa[b], s_hbm.at[pl.ds(base + i * CH, CH)], w[b])

        pltpu.make_async_copy(
            m_hbm.at[pl.ds(base, CH)], mbuf[b], ml[b]).wait()
        pltpu.sync_copy(mbuf[b], acc_sh.at[idxs.at[i]], add=True)
      return carry

    lax.fori_loop(0, nch // 2, outer, 0)
    pltpu.make_async_copy(bufa1, s_hbm.at[pl.ds(base, CH)], w1).wait()
    plsc.subcore_barrier()
    pltpu.sync_copy(acc_sh.at[pl.ds(sid * rpt, rpt)], tbuf)
    pltpu.sync_copy(
        tbuf, out_hbm.at[pl.ds(cid * NN + sid * rpt, rpt)])

  return k(tab_a, tab_b, rowg.reshape(-1, CH), colg.reshape(-1, CH),
           m, rows.reshape(-1, CH))


def _silu(x):
  return x * jax.nn.sigmoid(x)


def _tc_init_ab(noise, w0a, w0b):
  def body(n_ref, wa_ref, wb_ref, a_ref, b_ref):
    n = n_ref[...]
    a_ref[...] = n * wa_ref[...]
    b_ref[...] = n * wb_ref[...]

  return pl.pallas_call(
      body,
      out_shape=(jax.ShapeDtypeStruct((NN, HID), F32),
                 jax.ShapeDtypeStruct((NN, HID), F32)),
  )(noise, w0a, w0b)


def _tc_edge(s, attr, v, b0, w1t, b1):
  BE = 2048
  ne = s.shape[0]

  def body(s_ref, at_ref, v_ref, b0_ref, w1_ref, b1_ref, out_ref):
    u = s_ref[...] + at_ref[...] * v_ref[...] + b0_ref[...]
    u = _silu(u)
    m = jnp.dot(u, w1_ref[...], preferred_element_type=F32) + b1_ref[...]
    out_ref[...] = _silu(m)

  return pl.pallas_call(
      body,
      grid=(ne // BE,),
      in_specs=[
          pl.BlockSpec((BE, HID), lambda i: (i, 0)),
          pl.BlockSpec((BE, 1), lambda i: (i, 0)),
          pl.BlockSpec((1, HID), lambda i: (0, 0)),
          pl.BlockSpec((1, HID), lambda i: (0, 0)),
          pl.BlockSpec((HID, HID), lambda i: (0, 0)),
          pl.BlockSpec((1, HID), lambda i: (0, 0)),
      ],
      out_specs=pl.BlockSpec((BE, HID), lambda i: (i, 0)),
      out_shape=jax.ShapeDtypeStruct((ne, HID), F32),
  )(s, attr, v, b0, w1t, b1)


def _tc_node(h, aggps, wh, wa, bn0, wn1, bn1, proj_ws, proj_bs, *,
             first_layer, residual):
  n_out = len(proj_ws)
  n_agg = len(aggps)
  out_shapes = (jax.ShapeDtypeStruct((NN, HID), F32),) + tuple(
      jax.ShapeDtypeStruct((NN, w.shape[1]), F32) for w in proj_ws)

  def body(*refs):
    h_ref = refs[0]
    ag_refs = refs[1:1 + n_agg]
    wh_ref, wa_ref, bn0_ref, wn1_ref, bn1_ref = refs[1 + n_agg:6 + n_agg]
    pw = refs[6 + n_agg:6 + n_agg + n_out]
    pb = refs[6 + n_agg + n_out:6 + n_agg + 2 * n_out]
    h_out = refs[6 + n_agg + 2 * n_out]
    outs = refs[7 + n_agg + 2 * n_out:]
    agg = ag_refs[0][:NN, :] + ag_refs[0][NN:, :]
    for ar in ag_refs[1:]:
      agg = agg + ar[:NN, :] + ar[NN:, :]
    h = h_ref[...]
    if first_layer:
      t = h * wh_ref[...]
    else:
      t = jnp.dot(h, wh_ref[...], preferred_element_type=F32)
    t = t + jnp.dot(agg, wa_ref[...], preferred_element_type=F32) + bn0_ref[...]
    t = _silu(t)
    o = jnp.dot(t, wn1_ref[...], preferred_element_type=F32) + bn1_ref[...]
    if residual:
      o = o + h
    h_out[...] = o
    for i in range(n_out):
      outs[i][...] = (
          jnp.dot(o, pw[i][...], preferred_element_type=F32) + pb[i][...])

  return pl.pallas_call(
      body,
      out_shape=out_shapes,
  )(h, *aggps, wh, wa, bn0, wn1, bn1, *proj_ws, *proj_bs)


def _tc_decode(x, w, b):
  BR = 256

  def body(xb_ref, xf_ref, w_ref, b_ref, out_ref):
    i = pl.program_id(0)
    xb = xb_ref[...]
    xf = xf_ref[...]
    wv = w_ref[...]
    qf = jnp.sum(xf * xf * wv, axis=1)
    qb = jnp.sum(xb * xb * wv, axis=1)
    y = xf * wv
    cross = lax.dot_general(xb, y, (((1,), (1,)), ((), ())),
                            preferred_element_type=F32)
    logit = qb[:, None] + qf[None, :] - 2.0 * cross + b_ref[0, 0]
    rid = i * BR + lax.broadcasted_iota(jnp.int32, (BR, NN), 0)
    cid = lax.broadcasted_iota(jnp.int32, (BR, NN), 1)
    adj = jax.nn.sigmoid(logit)
    out_ref[...] = jnp.where(rid == cid, 0.0, adj)

  return pl.pallas_call(
      body,
      grid=(NN // BR,),
      in_specs=[
          pl.BlockSpec((BR, EMB), lambda i: (i, 0)),
          pl.BlockSpec((NN, EMB), lambda i: (0, 0)),
          pl.BlockSpec((1, EMB), lambda i: (0, 0)),
          pl.BlockSpec((1, 1), lambda i: (0, 0)),
      ],
      out_specs=pl.BlockSpec((BR, NN), lambda i: (i, 0)),
      out_shape=jax.ShapeDtypeStruct((NN, NN), F32),
  )(x, x, w, b)


def kernel(nodes, edges, edge_attr, params):
  row = edges[0]
  col = edges[1]
  noise = jax.random.normal(jax.random.key(1), (NN, 1), dtype=F32)

  gcl = [params["gcl_%d" % i] for i in range(NLAYERS)]
  e0 = []
  for i, g in enumerate(gcl):
    w = g["edge_mlp_0"]["W"]
    f = 1 if i == 0 else HID
    e0.append((w[:, :f].T, w[:, f:2 * f].T, w[:, 2 * f][None, :],
               g["edge_mlp_0"]["b"][None, :]))

  zero_b = jnp.zeros((1, HID), F32)
  a_tab, b_tab = _tc_init_ab(noise, e0[0][0], e0[0][1])

  ne2 = NE // 2
  rows = (row[:ne2], row[ne2:])
  cols = (col[:ne2], col[ne2:])
  attrs = (edge_attr[:ne2], edge_attr[ne2:])

  h = noise
  x = None
  for i in range(NLAYERS):
    g = gcl[i]
    w1t = g["edge_mlp_1"]["W"].T
    b1 = g["edge_mlp_1"]["b"][None, :]
    s0 = _sc_gather_add(a_tab, b_tab, rows[0], cols[0])
    m0 = _tc_edge(s0, attrs[0], e0[i][2], e0[i][3], w1t, b1)
    s1, p0 = _sc_gather_scatter(a_tab, b_tab, rows[1], cols[1], m0, rows[0])
    m1 = _tc_edge(s1, attrs[1], e0[i][2], e0[i][3], w1t, b1)
    p1 = _sc_segment_sum(m1, rows[1])
    aggps = [p0, p1]
    f = 1 if i == 0 else HID
    wh_full = g["node_mlp_0"]["W"]
    wh = wh_full[:, :f].T
    wa = wh_full[:, f:].T
    bn0 = g["node_mlp_0"]["b"][None, :]
    wn1 = g["node_mlp_1"]["W"].T
    bn1 = g["node_mlp_1"]["b"][None, :]
    if i < NLAYERS - 1:
      nxt_a, nxt_b = e0[i + 1][0], e0[i + 1][1]
      h, a_tab, b_tab = _tc_node(
          h, aggps, wh, wa, bn0, wn1, bn1,
          [nxt_a, nxt_b], [zero_b, zero_b],
          first_layer=(i == 0), residual=(i > 0))
    else:
      _, x = _tc_node(
          h, aggps, wh, wa, bn0, wn1, bn1,
          [params["fc_emb"]["W"].T], [params["fc_emb"]["b"][None, :]],
          first_layer=False, residual=True)

  adj = _tc_decode(x, params["fc_dec"]["W"], params["fc_dec"]["b"][None, :])
  return adj, x


# speedup vs baseline: 1.0605x; 1.0605x over previous
import functools

import jax
import jax.numpy as jnp
from jax import lax
from jax.experimental import pallas as pl
from jax.experimental.pallas import tpu as pltpu
from jax.experimental.pallas import tpu_sc as plsc

F32 = jnp.float32
NN = 2048
NE = 65536
HID = 128
EMB = 32
NLAYERS = 4
NC, NS = 2, 16
NW = NC * NS
EPT = NE // NW
CH = 128

_mesh = functools.partial(
    plsc.VectorSubcoreMesh, core_axis_name="c", subcore_axis_name="s")


def _sc_gather_add(tab_a, tab_b, row, col):
  ne = row.shape[0]
  ept = ne // NW
  nch = ept // CH

  @functools.partial(
      pl.kernel,
      out_type=jax.ShapeDtypeStruct((ne, HID), F32),
      mesh=_mesh(),
      scratch_types=[
          pltpu.VMEM((nch, CH), jnp.int32), pltpu.VMEM((nch, CH), jnp.int32),
          pltpu.VMEM((CH, HID), F32), pltpu.VMEM((CH, HID), F32),
          pltpu.VMEM((CH, HID), F32), pltpu.VMEM((CH, HID), F32),
          pltpu.VMEM((CH, HID), F32), pltpu.VMEM((CH, HID), F32),
          pltpu.SemaphoreType.DMA, pltpu.SemaphoreType.DMA,
          pltpu.SemaphoreType.DMA, pltpu.SemaphoreType.DMA,
          pltpu.SemaphoreType.DMA, pltpu.SemaphoreType.DMA,
      ],
  )
  def k(a_hbm, b_hbm, row_hbm, col_hbm, s_hbm,
        idxr, idxc, bufa0, bufa1, bufb0, bufb1,
        ubuf0, ubuf1, ga0, ga1, gb0, gb1, w0, w1):
    bufa = [bufa0, bufa1]
    bufb = [bufb0, bufb1]
    ubuf = [ubuf0, ubuf1]
    ga = [ga0, ga1]
    gb = [gb0, gb1]
    w = [w0, w1]
    wid = lax.axis_index("s") * NC + lax.axis_index("c")
    base = wid * ept

    pltpu.sync_copy(row_hbm.at[pl.ds(wid * nch, nch)], idxr)
    pltpu.sync_copy(col_hbm.at[pl.ds(wid * nch, nch)], idxc)

    def issue(i, b):
      pltpu.async_copy(a_hbm.at[idxr.at[i]], bufa[b], ga[b])
      pltpu.async_copy(b_hbm.at[idxc.at[i]], bufb[b], gb[b])

    issue(0, 0)

    def outer(j, carry):
      for b in (0, 1):
        i = 2 * j + b
        nb = 1 - b

        @pl.when(i + 1 < nch)
        def _():
          issue(i + 1, nb)

        pltpu.make_async_copy(a_hbm.at[idxr.at[0]], bufa[b], ga[b]).wait()
        pltpu.make_async_copy(b_hbm.at[idxc.at[0]], bufb[b], gb[b]).wait()

        @pl.when(i >= 2)
        def _():
          pltpu.make_async_copy(
              ubuf[b], s_hbm.at[pl.ds(base, CH)], w[b]).wait()

        def crow(r, c1):
          for c8 in range(HID // 16):
            sl = pl.ds(c8 * 16, 16)
            ubuf[b][r, sl] = bufa[b][r, sl] + bufb[b][r, sl]
          return c1

        lax.fori_loop(0, CH, crow, 0)
        pltpu.async_copy(ubuf[b], s_hbm.at[pl.ds(base + i * CH, CH)], w[b])
      return carry

    lax.fori_loop(0, nch // 2, outer, 0)
    pltpu.make_async_copy(ubuf0, s_hbm.at[pl.ds(base, CH)], w0).wait()
    pltpu.make_async_copy(ubuf1, s_hbm.at[pl.ds(base, CH)], w1).wait()

  return k(tab_a, tab_b, row.reshape(-1, CH), col.reshape(-1, CH))


def _sc_segment_sum(m, row):
  rpt = NN // NS
  ne = row.shape[0]
  ept = ne // NW

  @functools.partial(
      pl.kernel,
      out_type=jax.ShapeDtypeStruct((NC * NN, HID), F32),
      mesh=_mesh(),
      scratch_types=[
          pltpu.VMEM((ept // CH, CH), jnp.int32),
          pltpu.VMEM((CH, HID), F32), pltpu.VMEM((CH, HID), F32),
          pltpu.VMEM((rpt, HID), F32),
          pltpu.VMEM_SHARED((NN, HID), F32),
          pltpu.SemaphoreType.DMA, pltpu.SemaphoreType.DMA,
      ],
  )
  def k(m_hbm, row_hbm, out_hbm, idx, mbuf0, mbuf1, tbuf, acc_sh,
        ml0, ml1):
    mbuf = [mbuf0, mbuf1]
    ml = [ml0, ml1]
    cid = lax.axis_index("c")
    sid = lax.axis_index("s")
    wid = sid * NC + cid
    nch = ept // CH

    def zrow(r, carry):
      def zcol(c8, c2):
        tbuf[r, pl.ds(c8 * 16, 16)] = jnp.zeros((16,), F32)
        return c2
      return lax.fori_loop(0, HID // 16, zcol, carry)

    lax.fori_loop(0, rpt, zrow, 0)
    pltpu.sync_copy(tbuf, acc_sh.at[pl.ds(sid * rpt, rpt)])
    plsc.subcore_barrier()

    base = wid * ept
    pltpu.sync_copy(row_hbm.at[pl.ds(wid * nch, nch)], idx)

    def issue(i, b):
      pltpu.async_copy(m_hbm.at[pl.ds(base + i * CH, CH)], mbuf[b], ml[b])

    issue(0, 0)

    def outer(j, carry):
      for b in (0, 1):
        i = 2 * j + b
        nb = 1 - b

        @pl.when(i + 1 < nch)
        def _():
          issue(i + 1, nb)

        pltpu.make_async_copy(
            m_hbm.at[pl.ds(base, CH)], mbuf[b], ml[b]).wait()
        pltpu.sync_copy(mbuf[b], acc_sh.at[idx.at[i]], add=True)
      return carry

    lax.fori_loop(0, nch // 2, outer, 0)
    plsc.subcore_barrier()

    pltpu.sync_copy(acc_sh.at[pl.ds(sid * rpt, rpt)], tbuf)
    pltpu.sync_copy(tbuf, out_hbm.at[pl.ds(cid * NN + sid * rpt, rpt)])

  return k(m, row.reshape(-1, CH))


def _sc_gather_scatter(tab_a, tab_b, rowg, colg, m, rows):
  ne = rowg.shape[0]
  ept = ne // NW
  nch = ept // CH
  rpt = NN // NS

  @functools.partial(
      pl.kernel,
      out_type=(jax.ShapeDtypeStruct((ne, HID), F32),
                jax.ShapeDtypeStruct((NC * NN, HID), F32)),
      mesh=_mesh(),
      scratch_types=[
          pltpu.VMEM((nch, CH), jnp.int32), pltpu.VMEM((nch, CH), jnp.int32),
          pltpu.VMEM((nch, CH), jnp.int32),
          pltpu.VMEM((CH, HID), F32), pltpu.VMEM((CH, HID), F32),
          pltpu.VMEM((CH, HID), F32), pltpu.VMEM((CH, HID), F32),
          pltpu.VMEM((CH, HID), F32), pltpu.VMEM((CH, HID), F32),
          pltpu.VMEM_SHARED((NN, HID), F32),
          pltpu.SemaphoreType.DMA, pltpu.SemaphoreType.DMA,
          pltpu.SemaphoreType.DMA, pltpu.SemaphoreType.DMA,
          pltpu.SemaphoreType.DMA, pltpu.SemaphoreType.DMA,
          pltpu.SemaphoreType.DMA, pltpu.SemaphoreType.DMA,
      ],
  )
  def k(a_hbm, b_hbm, rowg_hbm, colg_hbm, m_hbm, rows_hbm, s_hbm, out_hbm,
        idxr, idxc, idxs, bufa0, bufa1, bufb0, bufb1, mbuf0, mbuf1,
        acc_sh, ga0, ga1, gb0, gb1, ml0, ml1, w0, w1):
    tbuf = bufb0
    bufa = [bufa0, bufa1]
    bufb = [bufb0, bufb1]
    mbuf = [mbuf0, mbuf1]
    ga = [ga0, ga1]
    gb = [gb0, gb1]
    ml = [ml0, ml1]
    w = [w0, w1]
    cid = lax.axis_index("c")
    sid = lax.axis_index("s")
    wid = sid * NC + cid
    base = wid * ept

    pltpu.sync_copy(rowg_hbm.at[pl.ds(wid * nch, nch)], idxr)
    pltpu.sync_copy(colg_hbm.at[pl.ds(wid * nch, nch)], idxc)
    pltpu.sync_copy(rows_hbm.at[pl.ds(wid * nch, nch)], idxs)

    def zrow(r, carry):
      def zcol(c8, c2):
        tbuf[r, pl.ds(c8 * 16, 16)] = jnp.zeros((16,), F32)
        return c2
      return lax.fori_loop(0, HID // 16, zcol, carry)

    lax.fori_loop(0, rpt, zrow, 0)
    pltpu.sync_copy(tbuf, acc_sh.at[pl.ds(sid * rpt, rpt)])
    plsc.subcore_barrier()

    def issue(i, b):
      pltpu.async_copy(a_hbm.at[idxr.at[i]], bufa[b], ga[b])
      pltpu.async_copy(b_hbm.at[idxc.at[i]], bufb[b], gb[b])
      pltpu.async_copy(m_hbm.at[pl.ds(base + i * CH, CH)], mbuf[b], ml[b])

    issue(0, 0)

    def outer(j, carry):
      for b in (0, 1):
        i = 2 * j + b
        nb = 1 - b

        @pl.when(i >= 1)
        def _():
          pltpu.make_async_copy(
              bufa[nb], s_hbm.at[pl.ds(base, CH)], w[nb]).wait()

        @pl.when(i + 1 < nch)
        def _():
          issue(i + 1, nb)

        pltpu.make_async_copy(a_hbm.at[idxr.at[0]], bufa[b], ga[b]).wait()
        pltpu.make_async_copy(b_hbm.at[idxc.at[0]], bufb[b], gb[b]).wait()

        def crow(r, c1):
          for c8 in range(HID // 16):
            sl = pl.ds(c8 * 16, 16)
            bufa[b][r, sl] = bufa[b][r, sl] + bufb[b][r, sl]
          return c1

        lax.fori_loop(0, CH, crow, 0)
        pltpu.async_copy(bufa[b], s_hbm.at[pl.ds(base + i * CH, CH)], w[b])

        pltpu.make_async_copy(
            m_hbm.at[pl.ds(base, CH)], mbuf[b], ml[b]).wait()
        pltpu.sync_copy(mbuf[b], acc_sh.at[idxs.at[i]], add=True)
      return carry

    lax.fori_loop(0, nch // 2, outer, 0)
    pltpu.make_async_copy(bufa1, s_hbm.at[pl.ds(base, CH)], w1).wait()
    plsc.subcore_barrier()
    pltpu.sync_copy(acc_sh.at[pl.ds(sid * rpt, rpt)], tbuf)
    pltpu.sync_copy(
        tbuf, out_hbm.at[pl.ds(cid * NN + sid * rpt, rpt)])

  return k(tab_a, tab_b, rowg.reshape(-1, CH), colg.reshape(-1, CH),
           m, rows.reshape(-1, CH))


def _silu(x):
  return x * jax.nn.sigmoid(x)


def _tc_init_ab(noise, w0a, w0b):
  def body(n_ref, wa_ref, wb_ref, a_ref, b_ref):
    n = n_ref[...]
    a_ref[...] = n * wa_ref[...]
    b_ref[...] = n * wb_ref[...]

  return pl.pallas_call(
      body,
      out_shape=(jax.ShapeDtypeStruct((NN, HID), F32),
                 jax.ShapeDtypeStruct((NN, HID), F32)),
  )(noise, w0a, w0b)


def _tc_edge(s, attr, v, b0, w1t, b1, *, half=None):
  BE = 2048
  ne = s.shape[0] if half is None else s.shape[0] // 2
  off = 0 if not half else ne // BE

  def body(s_ref, at_ref, v_ref, b0_ref, w1_ref, b1_ref, out_ref):
    u = s_ref[...] + at_ref[...] * v_ref[...] + b0_ref[...]
    u = _silu(u)
    m = jnp.dot(u, w1_ref[...], preferred_element_type=F32) + b1_ref[...]
    out_ref[...] = _silu(m)

  return pl.pallas_call(
      body,
      grid=(ne // BE,),
      in_specs=[
          pl.BlockSpec((BE, HID), lambda i: (i + off, 0)),
          pl.BlockSpec((BE, 1), lambda i: (i, 0)),
          pl.BlockSpec((1, HID), lambda i: (0, 0)),
          pl.BlockSpec((1, HID), lambda i: (0, 0)),
          pl.BlockSpec((HID, HID), lambda i: (0, 0)),
          pl.BlockSpec((1, HID), lambda i: (0, 0)),
      ],
      out_specs=pl.BlockSpec((BE, HID), lambda i: (i, 0)),
      out_shape=jax.ShapeDtypeStruct((ne, HID), F32),
  )(s, attr, v, b0, w1t, b1)


def _tc_node(h, aggps, wh, wa, bn0, wn1, bn1, proj_ws, proj_bs, *,
             first_layer, residual):
  n_out = len(proj_ws)
  n_agg = len(aggps)
  out_shapes = (jax.ShapeDtypeStruct((NN, HID), F32),) + tuple(
      jax.ShapeDtypeStruct((NN, w.shape[1]), F32) for w in proj_ws)

  def body(*refs):
    h_ref = refs[0]
    ag_refs = refs[1:1 + n_agg]
    wh_ref, wa_ref, bn0_ref, wn1_ref, bn1_ref = refs[1 + n_agg:6 + n_agg]
    pw = refs[6 + n_agg:6 + n_agg + n_out]
    pb = refs[6 + n_agg + n_out:6 + n_agg + 2 * n_out]
    h_out = refs[6 + n_agg + 2 * n_out]
    outs = refs[7 + n_agg + 2 * n_out:]
    agg = ag_refs[0][:NN, :] + ag_refs[0][NN:, :]
    for ar in ag_refs[1:]:
      agg = agg + ar[:NN, :] + ar[NN:, :]
    h = h_ref[...]
    if first_layer:
      t = h * wh_ref[...]
    else:
      t = jnp.dot(h, wh_ref[...], preferred_element_type=F32)
    t = t + jnp.dot(agg, wa_ref[...], preferred_element_type=F32) + bn0_ref[...]
    t = _silu(t)
    o = jnp.dot(t, wn1_ref[...], preferred_element_type=F32) + bn1_ref[...]
    if residual:
      o = o + h
    h_out[...] = o
    for i in range(n_out):
      outs[i][...] = (
          jnp.dot(o, pw[i][...], preferred_element_type=F32) + pb[i][...])

  return pl.pallas_call(
      body,
      out_shape=out_shapes,
  )(h, *aggps, wh, wa, bn0, wn1, bn1, *proj_ws, *proj_bs)


def _tc_decode(x, w, b):
  BR = 256

  def body(xb_ref, xf_ref, w_ref, b_ref, out_ref):
    i = pl.program_id(0)
    mu = jnp.mean(xf_ref[...], axis=0, keepdims=True)
    xb = xb_ref[...] - mu
    xf = xf_ref[...] - mu
    wv = w_ref[...]
    qf = jnp.sum(xf * xf * wv, axis=1)
    qb = jnp.sum(xb * xb * wv, axis=1)
    y = xf * wv
    cross = lax.dot_general(xb, y, (((1,), (1,)), ((), ())),
                            preferred_element_type=F32)
    logit = qb[:, None] + qf[None, :] - 2.0 * cross + b_ref[0, 0]
    rid = i * BR + lax.broadcasted_iota(jnp.int32, (BR, NN), 0)
    cid = lax.broadcasted_iota(jnp.int32, (BR, NN), 1)
    adj = jax.nn.sigmoid(logit)
    out_ref[...] = jnp.where(rid == cid, 0.0, adj)

  return pl.pallas_call(
      body,
      grid=(NN // BR,),
      in_specs=[
          pl.BlockSpec((BR, EMB), lambda i: (i, 0)),
          pl.BlockSpec((NN, EMB), lambda i: (0, 0)),
          pl.BlockSpec((1, EMB), lambda i: (0, 0)),
          pl.BlockSpec((1, 1), lambda i: (0, 0)),
      ],
      out_specs=pl.BlockSpec((BR, NN), lambda i: (i, 0)),
      out_shape=jax.ShapeDtypeStruct((NN, NN), F32),
  )(x, x, w, b)


def kernel(nodes, edges, edge_attr, params):
  row = edges[0]
  col = edges[1]
  noise = jax.random.normal(jax.random.key(1), (NN, 1), dtype=F32)

  gcl = [params["gcl_%d" % i] for i in range(NLAYERS)]
  e0 = []
  for i, g in enumerate(gcl):
    w = g["edge_mlp_0"]["W"]
    f = 1 if i == 0 else HID
    e0.append((w[:, :f].T, w[:, f:2 * f].T, w[:, 2 * f][None, :],
               g["edge_mlp_0"]["b"][None, :]))

  zero_b = jnp.zeros((1, HID), F32)
  a_tab, b_tab = _tc_init_ab(noise, e0[0][0], e0[0][1])

  ne2 = NE // 2
  rows = (row[:ne2], row[ne2:])
  cols = (col[:ne2], col[ne2:])
  attrs = (edge_attr[:ne2], edge_attr[ne2:])

  h = noise
  x = None
  for i in range(NLAYERS):
    g = gcl[i]
    w1t = g["edge_mlp_1"]["W"].T
    b1 = g["edge_mlp_1"]["b"][None, :]
    s = _sc_gather_add(a_tab, b_tab, row, col)
    ms = [_tc_edge(s, attrs[p], e0[i][2], e0[i][3], w1t, b1, half=p)
          for p in range(2)]
    aggps = [_sc_segment_sum(ms[p], rows[p]) for p in range(2)]
    f = 1 if i == 0 else HID
    wh_full = g["node_mlp_0"]["W"]
    wh = wh_full[:, :f].T
    wa = wh_full[:, f:].T
    bn0 = g["node_mlp_0"]["b"][None, :]
    wn1 = g["node_mlp_1"]["W"].T
    bn1 = g["node_mlp_1"]["b"][None, :]
    if i < NLAYERS - 1:
      nxt_a, nxt_b = e0[i + 1][0], e0[i + 1][1]
      h, a_tab, b_tab = _tc_node(
          h, aggps, wh, wa, bn0, wn1, bn1,
          [nxt_a, nxt_b], [zero_b, zero_b],
          first_layer=(i == 0), residual=(i > 0))
    else:
      _, x = _tc_node(
          h, aggps, wh, wa, bn0, wn1, bn1,
          [params["fc_emb"]["W"].T], [params["fc_emb"]["b"][None, :]],
          first_layer=False, residual=True)

  adj = _tc_decode(x, params["fc_dec"]["W"], params["fc_dec"]["b"][None, :])
  return adj, x


# speedup vs baseline: 1.0974x; 1.0347x over previous
import functools

import jax
import jax.numpy as jnp
from jax import lax
from jax.experimental import pallas as pl
from jax.experimental.pallas import tpu as pltpu
from jax.experimental.pallas import tpu_sc as plsc

F32 = jnp.float32
NN = 2048
NE = 65536
HID = 128
EMB = 32
NLAYERS = 4
NC, NS = 2, 16
NW = NC * NS
EPT = NE // NW
CH = 128

_mesh = functools.partial(
    plsc.VectorSubcoreMesh, core_axis_name="c", subcore_axis_name="s")


def _sc_gather_add(tab_a, tab_b, row, col):
  ne = row.shape[0]
  ept = ne // NW
  nch = ept // CH

  @functools.partial(
      pl.kernel,
      out_type=jax.ShapeDtypeStruct((ne, HID), F32),
      mesh=_mesh(),
      scratch_types=[
          pltpu.VMEM((nch, CH), jnp.int32), pltpu.VMEM((nch, CH), jnp.int32),
          pltpu.VMEM((CH, HID), F32), pltpu.VMEM((CH, HID), F32),
          pltpu.VMEM((CH, HID), F32), pltpu.VMEM((CH, HID), F32),
          pltpu.VMEM((CH, HID), F32), pltpu.VMEM((CH, HID), F32),
          pltpu.SemaphoreType.DMA, pltpu.SemaphoreType.DMA,
          pltpu.SemaphoreType.DMA, pltpu.SemaphoreType.DMA,
          pltpu.SemaphoreType.DMA, pltpu.SemaphoreType.DMA,
      ],
  )
  def k(a_hbm, b_hbm, row_hbm, col_hbm, s_hbm,
        idxr, idxc, bufa0, bufa1, bufb0, bufb1,
        ubuf0, ubuf1, ga0, ga1, gb0, gb1, w0, w1):
    bufa = [bufa0, bufa1]
    bufb = [bufb0, bufb1]
    ubuf = [ubuf0, ubuf1]
    ga = [ga0, ga1]
    gb = [gb0, gb1]
    w = [w0, w1]
    wid = lax.axis_index("s") * NC + lax.axis_index("c")
    base = wid * ept

    pltpu.sync_copy(row_hbm.at[pl.ds(wid * nch, nch)], idxr)
    pltpu.sync_copy(col_hbm.at[pl.ds(wid * nch, nch)], idxc)

    def issue(i, b):
      pltpu.async_copy(a_hbm.at[idxr.at[i]], bufa[b], ga[b])
      pltpu.async_copy(b_hbm.at[idxc.at[i]], bufb[b], gb[b])

    issue(0, 0)

    def outer(j, carry):
      for b in (0, 1):
        i = 2 * j + b
        nb = 1 - b

        @pl.when(i + 1 < nch)
        def _():
          issue(i + 1, nb)

        pltpu.make_async_copy(a_hbm.at[idxr.at[0]], bufa[b], ga[b]).wait()
        pltpu.make_async_copy(b_hbm.at[idxc.at[0]], bufb[b], gb[b]).wait()

        @pl.when(i >= 2)
        def _():
          pltpu.make_async_copy(
              ubuf[b], s_hbm.at[pl.ds(base, CH)], w[b]).wait()

        def crow(r, c1):
          for c8 in range(HID // 16):
            sl = pl.ds(c8 * 16, 16)
            ubuf[b][r, sl] = bufa[b][r, sl] + bufb[b][r, sl]
          return c1

        lax.fori_loop(0, CH, crow, 0)
        pltpu.async_copy(ubuf[b], s_hbm.at[pl.ds(base + i * CH, CH)], w[b])
      return carry

    lax.fori_loop(0, nch // 2, outer, 0)
    pltpu.make_async_copy(ubuf0, s_hbm.at[pl.ds(base, CH)], w0).wait()
    pltpu.make_async_copy(ubuf1, s_hbm.at[pl.ds(base, CH)], w1).wait()

  return k(tab_a, tab_b, row.reshape(-1, CH), col.reshape(-1, CH))


def _sc_segment_sum(m, row):
  rpt = NN // NS
  ne = row.shape[0]
  ept = ne // NW

  @functools.partial(
      pl.kernel,
      out_type=jax.ShapeDtypeStruct((NC * NN, HID), F32),
      mesh=_mesh(),
      scratch_types=[
          pltpu.VMEM((ept // CH, CH), jnp.int32),
          pltpu.VMEM((CH, HID), F32), pltpu.VMEM((CH, HID), F32),
          pltpu.VMEM((rpt, HID), F32),
          pltpu.VMEM_SHARED((NN, HID), F32),
          pltpu.SemaphoreType.DMA, pltpu.SemaphoreType.DMA,
      ],
  )
  def k(m_hbm, row_hbm, out_hbm, idx, mbuf0, mbuf1, tbuf, acc_sh,
        ml0, ml1):
    mbuf = [mbuf0, mbuf1]
    ml = [ml0, ml1]
    cid = lax.axis_index("c")
    sid = lax.axis_index("s")
    wid = sid * NC + cid
    nch = ept // CH

    def zrow(r, carry):
      def zcol(c8, c2):
        tbuf[r, pl.ds(c8 * 16, 16)] = jnp.zeros((16,), F32)
        return c2
      return lax.fori_loop(0, HID // 16, zcol, carry)

    lax.fori_loop(0, rpt, zrow, 0)
    pltpu.sync_copy(tbuf, acc_sh.at[pl.ds(sid * rpt, rpt)])
    plsc.subcore_barrier()

    base = wid * ept
    pltpu.sync_copy(row_hbm.at[pl.ds(wid * nch, nch)], idx)

    def issue(i, b):
      pltpu.async_copy(m_hbm.at[pl.ds(base + i * CH, CH)], mbuf[b], ml[b])

    issue(0, 0)

    def outer(j, carry):
      for b in (0, 1):
        i = 2 * j + b
        nb = 1 - b

        @pl.when(i + 1 < nch)
        def _():
          issue(i + 1, nb)

        pltpu.make_async_copy(
            m_hbm.at[pl.ds(base, CH)], mbuf[b], ml[b]).wait()
        pltpu.sync_copy(mbuf[b], acc_sh.at[idx.at[i]], add=True)
      return carry

    lax.fori_loop(0, nch // 2, outer, 0)
    plsc.subcore_barrier()

    pltpu.sync_copy(acc_sh.at[pl.ds(sid * rpt, rpt)], tbuf)
    pltpu.sync_copy(tbuf, out_hbm.at[pl.ds(cid * NN + sid * rpt, rpt)])

  return k(m, row.reshape(-1, CH))


def _sc_gather_scatter(tab_a, tab_b, rowg, colg, m, rows):
  ne = rowg.shape[0]
  ept = ne // NW
  nch = ept // CH
  rpt = NN // NS

  @functools.partial(
      pl.kernel,
      out_type=(jax.ShapeDtypeStruct((ne, HID), F32),
                jax.ShapeDtypeStruct((NC * NN, HID), F32)),
      mesh=_mesh(),
      scratch_types=[
          pltpu.VMEM((nch, CH), jnp.int32), pltpu.VMEM((nch, CH), jnp.int32),
          pltpu.VMEM((nch, CH), jnp.int32),
          pltpu.VMEM((CH, HID), F32), pltpu.VMEM((CH, HID), F32),
          pltpu.VMEM((CH, HID), F32), pltpu.VMEM((CH, HID), F32),
          pltpu.VMEM((CH, HID), F32), pltpu.VMEM((CH, HID), F32),
          pltpu.VMEM_SHARED((NN, HID), F32),
          pltpu.SemaphoreType.DMA, pltpu.SemaphoreType.DMA,
          pltpu.SemaphoreType.DMA, pltpu.SemaphoreType.DMA,
          pltpu.SemaphoreType.DMA, pltpu.SemaphoreType.DMA,
          pltpu.SemaphoreType.DMA, pltpu.SemaphoreType.DMA,
      ],
  )
  def k(a_hbm, b_hbm, rowg_hbm, colg_hbm, m_hbm, rows_hbm, s_hbm, out_hbm,
        idxr, idxc, idxs, bufa0, bufa1, bufb0, bufb1, mbuf0, mbuf1,
        acc_sh, ga0, ga1, gb0, gb1, ml0, ml1, w0, w1):
    tbuf = bufb0
    bufa = [bufa0, bufa1]
    bufb = [bufb0, bufb1]
    mbuf = [mbuf0, mbuf1]
    ga = [ga0, ga1]
    gb = [gb0, gb1]
    ml = [ml0, ml1]
    w = [w0, w1]
    cid = lax.axis_index("c")
    sid = lax.axis_index("s")
    wid = sid * NC + cid
    base = wid * ept

    pltpu.sync_copy(rowg_hbm.at[pl.ds(wid * nch, nch)], idxr)
    pltpu.sync_copy(colg_hbm.at[pl.ds(wid * nch, nch)], idxc)
    pltpu.sync_copy(rows_hbm.at[pl.ds(wid * nch, nch)], idxs)

    def zrow(r, carry):
      def zcol(c8, c2):
        tbuf[r, pl.ds(c8 * 16, 16)] = jnp.zeros((16,), F32)
        return c2
      return lax.fori_loop(0, HID // 16, zcol, carry)

    lax.fori_loop(0, rpt, zrow, 0)
    pltpu.sync_copy(tbuf, acc_sh.at[pl.ds(sid * rpt, rpt)])
    plsc.subcore_barrier()

    def issue(i, b):
      pltpu.async_copy(a_hbm.at[idxr.at[i]], bufa[b], ga[b])
      pltpu.async_copy(b_hbm.at[idxc.at[i]], bufb[b], gb[b])
      pltpu.async_copy(m_hbm.at[pl.ds(base + i * CH, CH)], mbuf[b], ml[b])

    issue(0, 0)

    def outer(j, carry):
      for b in (0, 1):
        i = 2 * j + b
        nb = 1 - b

        @pl.when(i >= 1)
        def _():
          pltpu.make_async_copy(
              bufa[nb], s_hbm.at[pl.ds(base, CH)], w[nb]).wait()

        @pl.when(i + 1 < nch)
        def _():
          issue(i + 1, nb)

        pltpu.make_async_copy(a_hbm.at[idxr.at[0]], bufa[b], ga[b]).wait()
        pltpu.make_async_copy(b_hbm.at[idxc.at[0]], bufb[b], gb[b]).wait()

        def crow(r, c1):
          for c8 in range(HID // 16):
            sl = pl.ds(c8 * 16, 16)
            bufa[b][r, sl] = bufa[b][r, sl] + bufb[b][r, sl]
          return c1

        lax.fori_loop(0, CH, crow, 0)
        pltpu.async_copy(bufa[b], s_hbm.at[pl.ds(base + i * CH, CH)], w[b])

        pltpu.make_async_copy(
            m_hbm.at[pl.ds(base, CH)], mbuf[b], ml[b]).wait()
        pltpu.sync_copy(mbuf[b], acc_sh.at[idxs.at[i]], add=True)
      return carry

    lax.fori_loop(0, nch // 2, outer, 0)
    pltpu.make_async_copy(bufa1, s_hbm.at[pl.ds(base, CH)], w1).wait()
    plsc.subcore_barrier()
    pltpu.sync_copy(acc_sh.at[pl.ds(sid * rpt, rpt)], tbuf)
    pltpu.sync_copy(
        tbuf, out_hbm.at[pl.ds(cid * NN + sid * rpt, rpt)])

  return k(tab_a, tab_b, rowg.reshape(-1, CH), colg.reshape(-1, CH),
           m, rows.reshape(-1, CH))


def _silu(x):
  return x * jax.nn.sigmoid(x)


def _tc_init_ab(noise, w0a, w0b):
  def body(n_ref, wa_ref, wb_ref, a_ref, b_ref):
    n = n_ref[...]
    a_ref[...] = n * wa_ref[...]
    b_ref[...] = n * wb_ref[...]

  return pl.pallas_call(
      body,
      out_shape=(jax.ShapeDtypeStruct((NN, HID), F32),
                 jax.ShapeDtypeStruct((NN, HID), F32)),
  )(noise, w0a, w0b)


def _tc_edge(s, attr, v, b0, w1t, b1, *, half=None):
  BE = 2048
  ne = s.shape[0] if half is None else s.shape[0] // 2
  off = 0 if not half else ne // BE

  def body(s_ref, at_ref, v_ref, b0_ref, w1_ref, b1_ref, out_ref):
    u = s_ref[...] + at_ref[...] * v_ref[...] + b0_ref[...]
    u = _silu(u)
    m = jnp.dot(u, w1_ref[...], preferred_element_type=F32) + b1_ref[...]
    out_ref[...] = _silu(m)

  return pl.pallas_call(
      body,
      grid=(ne // BE,),
      in_specs=[
          pl.BlockSpec((BE, HID), lambda i: (i + off, 0)),
          pl.BlockSpec((BE, 1), lambda i: (i, 0)),
          pl.BlockSpec((1, HID), lambda i: (0, 0)),
          pl.BlockSpec((1, HID), lambda i: (0, 0)),
          pl.BlockSpec((HID, HID), lambda i: (0, 0)),
          pl.BlockSpec((1, HID), lambda i: (0, 0)),
      ],
      out_specs=pl.BlockSpec((BE, HID), lambda i: (i, 0)),
      out_shape=jax.ShapeDtypeStruct((ne, HID), F32),
  )(s, attr, v, b0, w1t, b1)


def _tc_node(h, aggps, wh, wa, bn0, wn1, bn1, proj_ws, proj_bs, *,
             first_layer, residual):
  n_out = len(proj_ws)
  n_agg = len(aggps)
  out_shapes = (jax.ShapeDtypeStruct((NN, HID), F32),) + tuple(
      jax.ShapeDtypeStruct((NN, w.shape[1]), F32) for w in proj_ws)

  def body(*refs):
    h_ref = refs[0]
    ag_refs = refs[1:1 + n_agg]
    wh_ref, wa_ref, bn0_ref, wn1_ref, bn1_ref = refs[1 + n_agg:6 + n_agg]
    pw = refs[6 + n_agg:6 + n_agg + n_out]
    pb = refs[6 + n_agg + n_out:6 + n_agg + 2 * n_out]
    h_out = refs[6 + n_agg + 2 * n_out]
    outs = refs[7 + n_agg + 2 * n_out:]
    agg = ag_refs[0][:NN, :] + ag_refs[0][NN:, :]
    for ar in ag_refs[1:]:
      agg = agg + ar[:NN, :] + ar[NN:, :]
    h = h_ref[...]
    if first_layer:
      t = h * wh_ref[...]
    else:
      t = jnp.dot(h, wh_ref[...], preferred_element_type=F32)
    t = t + jnp.dot(agg, wa_ref[...], preferred_element_type=F32) + bn0_ref[...]
    t = _silu(t)
    o = jnp.dot(t, wn1_ref[...], preferred_element_type=F32) + bn1_ref[...]
    if residual:
      o = o + h
    h_out[...] = o
    for i in range(n_out):
      outs[i][...] = (
          jnp.dot(o, pw[i][...], preferred_element_type=F32) + pb[i][...])

  return pl.pallas_call(
      body,
      out_shape=out_shapes,
  )(h, *aggps, wh, wa, bn0, wn1, bn1, *proj_ws, *proj_bs)


def _tc_decode(x, w, b):
  BR = 256

  def body(xb_ref, xf_ref, w_ref, b_ref, out_ref):
    i = pl.program_id(0)
    mu = jnp.mean(xf_ref[...], axis=0, keepdims=True)
    xb = xb_ref[...] - mu
    xf = xf_ref[...] - mu
    wv = w_ref[...]
    qf = jnp.sum(xf * xf * wv, axis=1)
    qb = jnp.sum(xb * xb * wv, axis=1)
    y = xf * wv
    cross = lax.dot_general(xb, y, (((1,), (1,)), ((), ())),
                            preferred_element_type=F32)
    logit = qb[:, None] + qf[None, :] - 2.0 * cross + b_ref[0, 0]
    rid = i * BR + lax.broadcasted_iota(jnp.int32, (BR, NN), 0)
    cid = lax.broadcasted_iota(jnp.int32, (BR, NN), 1)
    adj = jax.nn.sigmoid(logit)
    out_ref[...] = jnp.where(rid == cid, 0.0, adj)

  return pl.pallas_call(
      body,
      grid=(NN // BR,),
      in_specs=[
          pl.BlockSpec((BR, EMB), lambda i: (i, 0)),
          pl.BlockSpec((NN, EMB), lambda i: (0, 0)),
          pl.BlockSpec((1, EMB), lambda i: (0, 0)),
          pl.BlockSpec((1, 1), lambda i: (0, 0)),
      ],
      out_specs=pl.BlockSpec((BR, NN), lambda i: (i, 0)),
      out_shape=jax.ShapeDtypeStruct((NN, NN), F32),
  )(x, x, w, b)


def kernel(nodes, edges, edge_attr, params):
  row = edges[0]
  col = edges[1]
  noise = jax.random.normal(jax.random.key(1), (NN, 1), dtype=F32)

  gcl = [params["gcl_%d" % i] for i in range(NLAYERS)]
  e0 = []
  for i, g in enumerate(gcl):
    w = g["edge_mlp_0"]["W"]
    f = 1 if i == 0 else HID
    e0.append((w[:, :f].T, w[:, f:2 * f].T, w[:, 2 * f][None, :],
               g["edge_mlp_0"]["b"][None, :]))

  zero_b = jnp.zeros((1, HID), F32)
  a_tab, b_tab = _tc_init_ab(noise, e0[0][0], e0[0][1])

  ne2 = NE // 2
  rows = (row[:ne2], row[ne2:])
  cols = (col[:ne2], col[ne2:])
  attrs = (edge_attr[:ne2], edge_attr[ne2:])

  h = noise
  x = None
  for i in range(NLAYERS):
    g = gcl[i]
    w1t = g["edge_mlp_1"]["W"].T
    b1 = g["edge_mlp_1"]["b"][None, :]
    ss = [_sc_gather_add(a_tab, b_tab, rows[p], cols[p]) for p in range(2)]
    ms = [_tc_edge(ss[p], attrs[p], e0[i][2], e0[i][3], w1t, b1)
          for p in range(2)]
    aggps = [_sc_segment_sum(ms[p], rows[p]) for p in range(2)]
    f = 1 if i == 0 else HID
    wh_full = g["node_mlp_0"]["W"]
    wh = wh_full[:, :f].T
    wa = wh_full[:, f:].T
    bn0 = g["node_mlp_0"]["b"][None, :]
    wn1 = g["node_mlp_1"]["W"].T
    bn1 = g["node_mlp_1"]["b"][None, :]
    if i < NLAYERS - 1:
      nxt_a, nxt_b = e0[i + 1][0], e0[i + 1][1]
      h, a_tab, b_tab = _tc_node(
          h, aggps, wh, wa, bn0, wn1, bn1,
          [nxt_a, nxt_b], [zero_b, zero_b],
          first_layer=(i == 0), residual=(i > 0))
    else:
      _, x = _tc_node(
          h, aggps, wh, wa, bn0, wn1, bn1,
          [params["fc_emb"]["W"].T], [params["fc_emb"]["b"][None, :]],
          first_layer=False, residual=True)

  adj = _tc_decode(x, params["fc_dec"]["W"], params["fc_dec"]["b"][None, :])
  return adj, x


# speedup vs baseline: 1.1119x; 1.0132x over previous
import functools

import jax
import jax.numpy as jnp
from jax import lax
from jax.experimental import pallas as pl
from jax.experimental.pallas import tpu as pltpu
from jax.experimental.pallas import tpu_sc as plsc

F32 = jnp.float32
NN = 2048
NE = 65536
HID = 128
EMB = 32
NLAYERS = 4
NC, NS = 2, 16
NW = NC * NS
EPT = NE // NW
CH = 128

_mesh = functools.partial(
    plsc.VectorSubcoreMesh, core_axis_name="c", subcore_axis_name="s")


def _sc_gather_add(tab_a, tab_b, row, col):
  ne = row.shape[0]
  ept = ne // NW
  nch = ept // CH

  @functools.partial(
      pl.kernel,
      out_type=jax.ShapeDtypeStruct((ne, HID), F32),
      mesh=_mesh(),
      scratch_types=[
          pltpu.VMEM((nch, CH), jnp.int32), pltpu.VMEM((nch, CH), jnp.int32),
          pltpu.VMEM((CH, HID), F32), pltpu.VMEM((CH, HID), F32),
          pltpu.VMEM((CH, HID), F32), pltpu.VMEM((CH, HID), F32),
          pltpu.VMEM((CH, HID), F32), pltpu.VMEM((CH, HID), F32),
          pltpu.SemaphoreType.DMA, pltpu.SemaphoreType.DMA,
          pltpu.SemaphoreType.DMA, pltpu.SemaphoreType.DMA,
          pltpu.SemaphoreType.DMA, pltpu.SemaphoreType.DMA,
      ],
  )
  def k(a_hbm, b_hbm, row_hbm, col_hbm, s_hbm,
        idxr, idxc, bufa0, bufa1, bufb0, bufb1,
        ubuf0, ubuf1, ga0, ga1, gb0, gb1, w0, w1):
    bufa = [bufa0, bufa1]
    bufb = [bufb0, bufb1]
    ubuf = [ubuf0, ubuf1]
    ga = [ga0, ga1]
    gb = [gb0, gb1]
    w = [w0, w1]
    wid = lax.axis_index("s") * NC + lax.axis_index("c")
    base = wid * ept

    pltpu.sync_copy(row_hbm.at[pl.ds(wid * nch, nch)], idxr)
    pltpu.sync_copy(col_hbm.at[pl.ds(wid * nch, nch)], idxc)

    def issue(i, b):
      pltpu.async_copy(a_hbm.at[idxr.at[i]], bufa[b], ga[b])
      pltpu.async_copy(b_hbm.at[idxc.at[i]], bufb[b], gb[b])

    issue(0, 0)

    def outer(j, carry):
      for b in (0, 1):
        i = 2 * j + b
        nb = 1 - b

        @pl.when(i + 1 < nch)
        def _():
          issue(i + 1, nb)

        pltpu.make_async_copy(a_hbm.at[idxr.at[0]], bufa[b], ga[b]).wait()
        pltpu.make_async_copy(b_hbm.at[idxc.at[0]], bufb[b], gb[b]).wait()

        @pl.when(i >= 2)
        def _():
          pltpu.make_async_copy(
              ubuf[b], s_hbm.at[pl.ds(base, CH)], w[b]).wait()

        def crow(r, c1):
          for c8 in range(HID // 16):
            sl = pl.ds(c8 * 16, 16)
            ubuf[b][r, sl] = bufa[b][r, sl] + bufb[b][r, sl]
          return c1

        lax.fori_loop(0, CH, crow, 0)
        pltpu.async_copy(ubuf[b], s_hbm.at[pl.ds(base + i * CH, CH)], w[b])
      return carry

    lax.fori_loop(0, nch // 2, outer, 0)
    pltpu.make_async_copy(ubuf0, s_hbm.at[pl.ds(base, CH)], w0).wait()
    pltpu.make_async_copy(ubuf1, s_hbm.at[pl.ds(base, CH)], w1).wait()

  return k(tab_a, tab_b, row.reshape(-1, CH), col.reshape(-1, CH))


def _sc_segment_sum(m, row):
  rpt = NN // NS
  ne = row.shape[0]
  ept = ne // NW

  @functools.partial(
      pl.kernel,
      out_type=jax.ShapeDtypeStruct((NC * NN, HID), F32),
      mesh=_mesh(),
      scratch_types=[
          pltpu.VMEM((ept // CH, CH), jnp.int32),
          pltpu.VMEM((CH, HID), F32), pltpu.VMEM((CH, HID), F32),
          pltpu.VMEM((rpt, HID), F32),
          pltpu.VMEM_SHARED((NN, HID), F32),
          pltpu.SemaphoreType.DMA, pltpu.SemaphoreType.DMA,
      ],
  )
  def k(m_hbm, row_hbm, out_hbm, idx, mbuf0, mbuf1, tbuf, acc_sh,
        ml0, ml1):
    mbuf = [mbuf0, mbuf1]
    ml = [ml0, ml1]
    cid = lax.axis_index("c")
    sid = lax.axis_index("s")
    wid = sid * NC + cid
    nch = ept // CH
    base = wid * ept

    def issue(i, b):
      pltpu.async_copy(m_hbm.at[pl.ds(base + i * CH, CH)], mbuf[b], ml[b])

    issue(0, 0)
    pltpu.sync_copy(row_hbm.at[pl.ds(wid * nch, nch)], idx)

    def zrow(r, carry):
      def zcol(c8, c2):
        tbuf[r, pl.ds(c8 * 16, 16)] = jnp.zeros((16,), F32)
        return c2
      return lax.fori_loop(0, HID // 16, zcol, carry)

    lax.fori_loop(0, rpt, zrow, 0)
    pltpu.sync_copy(tbuf, acc_sh.at[pl.ds(sid * rpt, rpt)])
    plsc.subcore_barrier()

    def outer(j, carry):
      for b in (0, 1):
        i = 2 * j + b
        nb = 1 - b

        @pl.when(i + 1 < nch)
        def _():
          issue(i + 1, nb)

        pltpu.make_async_copy(
            m_hbm.at[pl.ds(base, CH)], mbuf[b], ml[b]).wait()
        pltpu.sync_copy(mbuf[b], acc_sh.at[idx.at[i]], add=True)
      return carry

    lax.fori_loop(0, nch // 2, outer, 0)
    plsc.subcore_barrier()

    pltpu.sync_copy(acc_sh.at[pl.ds(sid * rpt, rpt)], tbuf)
    pltpu.sync_copy(tbuf, out_hbm.at[pl.ds(cid * NN + sid * rpt, rpt)])

  return k(m, row.reshape(-1, CH))


def _silu(x):
  return x * jax.nn.sigmoid(x)


def _tc_init_ab(noise, w0a, w0b):
  def body(n_ref, wa_ref, wb_ref, a_ref, b_ref):
    n = n_ref[...]
    a_ref[...] = n * wa_ref[...]
    b_ref[...] = n * wb_ref[...]

  return pl.pallas_call(
      body,
      out_shape=(jax.ShapeDtypeStruct((NN, HID), F32),
                 jax.ShapeDtypeStruct((NN, HID), F32)),
  )(noise, w0a, w0b)


def _tc_edge(s, attr, v, b0, w1t, b1, *, half=None):
  BE = 2048
  ne = s.shape[0] if half is None else s.shape[0] // 2
  off = 0 if not half else ne // BE

  def body(s_ref, at_ref, v_ref, b0_ref, w1_ref, b1_ref, out_ref):
    u = s_ref[...] + at_ref[...] * v_ref[...] + b0_ref[...]
    u = _silu(u)
    m = jnp.dot(u, w1_ref[...], preferred_element_type=F32) + b1_ref[...]
    out_ref[...] = _silu(m)

  return pl.pallas_call(
      body,
      grid=(ne // BE,),
      in_specs=[
          pl.BlockSpec((BE, HID), lambda i: (i + off, 0)),
          pl.BlockSpec((BE, 1), lambda i: (i, 0)),
          pl.BlockSpec((1, HID), lambda i: (0, 0)),
          pl.BlockSpec((1, HID), lambda i: (0, 0)),
          pl.BlockSpec((HID, HID), lambda i: (0, 0)),
          pl.BlockSpec((1, HID), lambda i: (0, 0)),
      ],
      out_specs=pl.BlockSpec((BE, HID), lambda i: (i, 0)),
      out_shape=jax.ShapeDtypeStruct((ne, HID), F32),
  )(s, attr, v, b0, w1t, b1)


def _tc_node(h, aggps, wh, wa, bn0, wn1, bn1, proj_ws, proj_bs, *,
             first_layer, residual):
  n_out = len(proj_ws)
  n_agg = len(aggps)
  out_shapes = (jax.ShapeDtypeStruct((NN, HID), F32),) + tuple(
      jax.ShapeDtypeStruct((NN, w.shape[1]), F32) for w in proj_ws)

  def body(*refs):
    h_ref = refs[0]
    ag_refs = refs[1:1 + n_agg]
    wh_ref, wa_ref, bn0_ref, wn1_ref, bn1_ref = refs[1 + n_agg:6 + n_agg]
    pw = refs[6 + n_agg:6 + n_agg + n_out]
    pb = refs[6 + n_agg + n_out:6 + n_agg + 2 * n_out]
    h_out = refs[6 + n_agg + 2 * n_out]
    outs = refs[7 + n_agg + 2 * n_out:]
    agg = ag_refs[0][:NN, :] + ag_refs[0][NN:, :]
    for ar in ag_refs[1:]:
      agg = agg + ar[:NN, :] + ar[NN:, :]
    h = h_ref[...]
    if first_layer:
      t = h * wh_ref[...]
    else:
      t = jnp.dot(h, wh_ref[...], preferred_element_type=F32)
    t = t + jnp.dot(agg, wa_ref[...], preferred_element_type=F32) + bn0_ref[...]
    t = _silu(t)
    o = jnp.dot(t, wn1_ref[...], preferred_element_type=F32) + bn1_ref[...]
    if residual:
      o = o + h
    h_out[...] = o
    for i in range(n_out):
      outs[i][...] = (
          jnp.dot(o, pw[i][...], preferred_element_type=F32) + pb[i][...])

  return pl.pallas_call(
      body,
      out_shape=out_shapes,
  )(h, *aggps, wh, wa, bn0, wn1, bn1, *proj_ws, *proj_bs)


def _tc_decode(x, w, b):
  BR = 256

  def body(xb_ref, xf_ref, w_ref, b_ref, out_ref):
    i = pl.program_id(0)
    mu = jnp.mean(xf_ref[...], axis=0, keepdims=True)
    xb = xb_ref[...] - mu
    xf = xf_ref[...] - mu
    wv = w_ref[...]
    qf = jnp.sum(xf * xf * wv, axis=1)
    qb = jnp.sum(xb * xb * wv, axis=1)
    y = xf * wv
    cross = lax.dot_general(xb, y, (((1,), (1,)), ((), ())),
                            preferred_element_type=F32)
    logit = qb[:, None] + qf[None, :] - 2.0 * cross + b_ref[0, 0]
    rid = i * BR + lax.broadcasted_iota(jnp.int32, (BR, NN), 0)
    cid = lax.broadcasted_iota(jnp.int32, (BR, NN), 1)
    adj = jax.nn.sigmoid(logit)
    out_ref[...] = jnp.where(rid == cid, 0.0, adj)

  return pl.pallas_call(
      body,
      grid=(NN // BR,),
      in_specs=[
          pl.BlockSpec((BR, EMB), lambda i: (i, 0)),
          pl.BlockSpec((NN, EMB), lambda i: (0, 0)),
          pl.BlockSpec((1, EMB), lambda i: (0, 0)),
          pl.BlockSpec((1, 1), lambda i: (0, 0)),
      ],
      out_specs=pl.BlockSpec((BR, NN), lambda i: (i, 0)),
      out_shape=jax.ShapeDtypeStruct((NN, NN), F32),
  )(x, x, w, b)


def kernel(nodes, edges, edge_attr, params):
  row = edges[0]
  col = edges[1]
  noise = jax.random.normal(jax.random.key(1), (NN, 1), dtype=F32)

  gcl = [params["gcl_%d" % i] for i in range(NLAYERS)]
  e0 = []
  for i, g in enumerate(gcl):
    w = g["edge_mlp_0"]["W"]
    f = 1 if i == 0 else HID
    e0.append((w[:, :f].T, w[:, f:2 * f].T, w[:, 2 * f][None, :],
               g["edge_mlp_0"]["b"][None, :]))

  zero_b = jnp.zeros((1, HID), F32)
  a_tab, b_tab = _tc_init_ab(noise, e0[0][0], e0[0][1])

  ne2 = NE // 2
  rows = (row[:ne2], row[ne2:])
  cols = (col[:ne2], col[ne2:])
  attrs = (edge_attr[:ne2], edge_attr[ne2:])

  h = noise
  x = None
  for i in range(NLAYERS):
    g = gcl[i]
    w1t = g["edge_mlp_1"]["W"].T
    b1 = g["edge_mlp_1"]["b"][None, :]
    ss = [_sc_gather_add(a_tab, b_tab, rows[p], cols[p]) for p in range(2)]
    ms = [_tc_edge(ss[p], attrs[p], e0[i][2], e0[i][3], w1t, b1)
          for p in range(2)]
    aggps = [_sc_segment_sum(ms[p], rows[p]) for p in range(2)]
    f = 1 if i == 0 else HID
    wh_full = g["node_mlp_0"]["W"]
    wh = wh_full[:, :f].T
    wa = wh_full[:, f:].T
    bn0 = g["node_mlp_0"]["b"][None, :]
    wn1 = g["node_mlp_1"]["W"].T
    bn1 = g["node_mlp_1"]["b"][None, :]
    if i < NLAYERS - 1:
      nxt_a, nxt_b = e0[i + 1][0], e0[i + 1][1]
      h, a_tab, b_tab = _tc_node(
          h, aggps, wh, wa, bn0, wn1, bn1,
          [nxt_a, nxt_b], [zero_b, zero_b],
          first_layer=(i == 0), residual=(i > 0))
    else:
      _, x = _tc_node(
          h, aggps, wh, wa, bn0, wn1, bn1,
          [params["fc_emb"]["W"].T], [params["fc_emb"]["b"][None, :]],
          first_layer=False, residual=True)

  adj = _tc_decode(x, params["fc_dec"]["W"], params["fc_dec"]["b"][None, :])
  return adj, x
